# Initial kernel scaffold; baseline (speedup 1.0000x reference)
import functools

import jax
import jax.numpy as jnp
from jax import lax
from jax.experimental import pallas as pl
from jax.experimental.pallas import tpu as pltpu
from jax.experimental.pallas import tpu_sc as plsc

N = 10000
D = 128
E = 320000
NC = 2
NS = 16
L = 16
NW = NC * NS
CHUNK = 128
NCHUNKS = E // CHUNK
ROWS_PER_TILE = N // NS
P1_BASE = NCHUNKS // NS
P1_EXTRA = NCHUNKS - P1_BASE * NS
P2_BASE = NCHUNKS // NW
P2_EXTRA = NCHUNKS - P2_BASE * NW


def _matmul_tc(x, W):
  def body(x_ref, w_ref, o_ref):
    o_ref[...] = lax.dot_general(
        x_ref[...], w_ref[...], (((1,), (1,)), ((), ())),
        preferred_element_type=jnp.float32)
  return pl.pallas_call(
      body, out_shape=jax.ShapeDtypeStruct((N, D), jnp.float32))(x, W)


def _finish_tc(parts, x, b, gamma, beta):
  def body(p_ref, x_ref, b_ref, g_ref, be_ref, o_ref):
    agg = p_ref[0] + p_ref[1] + b_ref[...]
    mean = jnp.mean(agg, axis=-1, keepdims=True)
    var = jnp.mean((agg - mean) ** 2, axis=-1, keepdims=True)
    h = g_ref[...] * (agg - mean) * lax.rsqrt(var + 1e-5) + be_ref[...]
    h = jax.nn.gelu(h, approximate=False)
    o_ref[...] = x_ref[...] + h
  return pl.pallas_call(
      body, out_shape=jax.ShapeDtypeStruct((N, D), jnp.float32))(
          parts, x, b, gamma, beta)


def _newton_rsqrt(xv):
  i = plsc.bitcast(xv, jnp.int32)
  yi = jnp.int32(0x5F3759DF) - lax.shift_right_logical(i, 1)
  y = plsc.bitcast(yi, jnp.float32)
  half = xv * 0.5
  for _ in range(3):
    y = y * (1.5 - half * y * y)
  return jnp.where(xv > 0.0, y, 0.0)


def _sc_aggregate(xw, row, col, ew, zn, znd):
  mesh = plsc.VectorSubcoreMesh(core_axis_name="c", subcore_axis_name="s")

  @functools.partial(
      pl.kernel,
      out_type=jax.ShapeDtypeStruct((NC, N, D), jnp.float32),
      mesh=mesh,
      scratch_types=[
          pltpu.VMEM((CHUNK,), jnp.int32),
          pltpu.VMEM((CHUNK,), jnp.int32),
          pltpu.VMEM((CHUNK,), jnp.float32),
          pltpu.VMEM((CHUNK,), jnp.float32),
          pltpu.VMEM((CHUNK, D), jnp.float32),
          pltpu.VMEM((N,), jnp.float32),
          pltpu.VMEM_SHARED((N,), jnp.float32),
          pltpu.VMEM_SHARED((N, D), jnp.float32),
          pltpu.SemaphoreType.DMA,
      ],
  )
  def k(xw_hbm, row_hbm, col_hbm, ew_hbm, zn_hbm, znd_hbm, out_hbm,
        idx_r, idx_c, ew_v, norm_v, rows_v, dis_v, deg_acc, agg_acc, sem):
    c = lax.axis_index("c")
    s = lax.axis_index("s")
    wid = s * NC + c

    @pl.when(s == 0)
    def _():
      pltpu.sync_copy(zn_hbm, deg_acc)
    pltpu.sync_copy(znd_hbm.at[pl.ds(s * ROWS_PER_TILE, ROWS_PER_TILE)],
                    agg_acc.at[pl.ds(s * ROWS_PER_TILE, ROWS_PER_TILE)])
    plsc.subcore_barrier()

    n1 = P1_BASE + jnp.where(s < P1_EXTRA, 1, 0)

    def deg_body(kk, _):
      off = (kk * NS + s) * CHUNK
      pltpu.sync_copy(col_hbm.at[pl.ds(off, CHUNK)], idx_c)
      pltpu.sync_copy(ew_hbm.at[pl.ds(off, CHUNK)], ew_v)
      pltpu.sync_copy(ew_v, deg_acc.at[idx_c], add=True)
      return ()

    lax.fori_loop(0, n1, deg_body, ())
    plsc.subcore_barrier()

    pltpu.sync_copy(deg_acc, dis_v)

    def rsqrt_body(i, _):
      sl = pl.ds(i * L, L)
      dis_v[sl] = _newton_rsqrt(dis_v[sl])
      return ()

    lax.fori_loop(0, N // L, rsqrt_body, ())

    n2 = P2_BASE + jnp.where(wid < P2_EXTRA, 1, 0)

    def agg_body(kk, _):
      off = (kk * NW + wid) * CHUNK
      pltpu.sync_copy(row_hbm.at[pl.ds(off, CHUNK)], idx_r)
      pltpu.sync_copy(col_hbm.at[pl.ds(off, CHUNK)], idx_c)
      pltpu.sync_copy(ew_hbm.at[pl.ds(off, CHUNK)], ew_v)
      pltpu.async_copy(xw_hbm.at[idx_r], rows_v, sem).wait()

      def norm_body(j, _):
        sl = pl.ds(j * L, L)
        dr = plsc.load_gather(dis_v, [idx_r[sl]])
        dc = plsc.load_gather(dis_v, [idx_c[sl]])
        norm_v[sl] = dr * ew_v[sl] * dc
        return ()

      lax.fori_loop(0, CHUNK // L, norm_body, ())

      def scale_body(r, _):
        nv = jnp.full((L,), norm_v[r], jnp.float32)
        for jj in range(D // L):
          sl = pl.ds(jj * L, L)
          rows_v[r, sl] = rows_v[r, sl] * nv
        return ()

      lax.fori_loop(0, CHUNK, scale_body, ())
      pltpu.sync_copy(rows_v, agg_acc.at[idx_c], add=True)
      return ()

    lax.fori_loop(0, n2, agg_body, ())
    plsc.subcore_barrier()

    sl = pl.ds(s * ROWS_PER_TILE, ROWS_PER_TILE)
    pltpu.sync_copy(agg_acc.at[sl], out_hbm.at[c, sl])

  return k(xw, row, col, ew, zn, znd)


def kernel(x, edge_index, edge_weight, W, b, gamma, beta):
  row = edge_index[0].astype(jnp.int32)
  col = edge_index[1].astype(jnp.int32)
  ew = edge_weight.astype(jnp.float32)
  xw = _matmul_tc(x, W)
  zn = jnp.zeros((N,), jnp.float32)
  znd = jnp.zeros((N, D), jnp.float32)
  parts = _sc_aggregate(xw, row, col, ew, zn, znd)
  return _finish_tc(parts, x, b.reshape(1, D), gamma.reshape(1, D),
                    beta.reshape(1, D))


# speedup vs baseline: 10.9648x; 10.9648x over previous
import functools

import jax
import jax.numpy as jnp
from jax import lax
from jax.experimental import pallas as pl
from jax.experimental.pallas import tpu as pltpu
from jax.experimental.pallas import tpu_sc as plsc

N = 10000
D = 128
E = 320000
NC = 2
NS = 16
L = 16
NW = NC * NS
CHUNK = 128
NCHUNKS = E // CHUNK
RPT = (N // NS) // 8 * 8
RPT_REM = N - NS * RPT
P1_BASE = NCHUNKS // NS
P1_EXTRA = NCHUNKS - P1_BASE * NS
P2_BASE = NCHUNKS // NW
P2_EXTRA = NCHUNKS - P2_BASE * NW


def _matmul_tc(x, W):
  def body(x_ref, w_ref, o_ref):
    o_ref[...] = lax.dot_general(
        x_ref[...], w_ref[...], (((1,), (1,)), ((), ())),
        preferred_element_type=jnp.float32)
  return pl.pallas_call(
      body, out_shape=jax.ShapeDtypeStruct((N, D), jnp.float32))(x, W)


def _finish_tc(parts, x, b, gamma, beta):
  def body(p_ref, x_ref, b_ref, g_ref, be_ref, o_ref):
    agg = p_ref[0] + p_ref[1] + b_ref[...]
    mean = jnp.mean(agg, axis=-1, keepdims=True)
    var = jnp.mean((agg - mean) ** 2, axis=-1, keepdims=True)
    h = g_ref[...] * (agg - mean) * lax.rsqrt(var + 1e-5) + be_ref[...]
    h = 0.5 * h * (1.0 + lax.erf(h * (1.0 / jnp.sqrt(2.0)).astype(jnp.float32)))
    o_ref[...] = x_ref[...] + h
  return pl.pallas_call(
      body, out_shape=jax.ShapeDtypeStruct((N, D), jnp.float32))(
          parts, x, b, gamma, beta)


def _newton_rsqrt(xv):
  i = lax.bitcast_convert_type(xv, jnp.int32)
  yi = jnp.int32(0x5F3759DF) - lax.shift_right_logical(i, 1)
  y = lax.bitcast_convert_type(yi, jnp.float32)
  half = xv * 0.5
  for _ in range(3):
    y = y * (1.5 - half * y * y)
  return jnp.where(xv > 0.0, y, 0.0)


def _sc_aggregate(xw, row, col, ew, zn, znd):
  mesh = plsc.VectorSubcoreMesh(core_axis_name="c", subcore_axis_name="s")

  @functools.partial(
      pl.kernel,
      out_type=jax.ShapeDtypeStruct((NC, N, D), jnp.float32),
      mesh=mesh,
      compiler_params=pltpu.CompilerParams(needs_layout_passes=False),
      scratch_types=[
          pltpu.VMEM((CHUNK,), jnp.int32),
          pltpu.VMEM((CHUNK,), jnp.int32),
          pltpu.VMEM((CHUNK,), jnp.float32),
          pltpu.VMEM((CHUNK,), jnp.float32),
          pltpu.VMEM((CHUNK, D), jnp.float32),
          pltpu.VMEM((N,), jnp.float32),
          pltpu.VMEM_SHARED((N,), jnp.float32),
          pltpu.VMEM_SHARED((N, D), jnp.float32),
          pltpu.SemaphoreType.DMA,
      ],
  )
  def k(xw_hbm, row_hbm, col_hbm, ew_hbm, zn_hbm, znd_hbm, out_hbm,
        idx_r, idx_c, ew_v, norm_v, rows_v, dis_v, deg_acc, agg_acc, sem):
    c = lax.axis_index("c")
    s = lax.axis_index("s")
    wid = s * NC + c

    @pl.when(s == 0)
    def _():
      pltpu.sync_copy(zn_hbm, deg_acc)
    sl0 = pl.ds(s * RPT, RPT)
    pltpu.sync_copy(znd_hbm.at[sl0], agg_acc.at[sl0])

    @pl.when(s == NS - 1)
    def _():
      slr = pl.ds(NS * RPT, RPT_REM)
      pltpu.sync_copy(znd_hbm.at[slr], agg_acc.at[slr])

    plsc.subcore_barrier()

    n1 = P1_BASE + jnp.where(s < P1_EXTRA, 1, 0)

    def deg_body(kk, _):
      off = (kk * NS + s) * CHUNK
      pltpu.sync_copy(col_hbm.at[pl.ds(off, CHUNK)], idx_c)
      pltpu.sync_copy(ew_hbm.at[pl.ds(off, CHUNK)], ew_v)
      pltpu.sync_copy(ew_v, deg_acc.at[idx_c], add=True)
      return ()

    lax.fori_loop(0, n1, deg_body, ())
    plsc.subcore_barrier()

    pltpu.sync_copy(deg_acc, dis_v)

    def rsqrt_body(i, _):
      sl = pl.ds(i * L, L)
      dis_v[sl] = _newton_rsqrt(dis_v[sl])
      return ()

    lax.fori_loop(0, N // L, rsqrt_body, ())

    n2 = P2_BASE + jnp.where(wid < P2_EXTRA, 1, 0)

    def agg_body(kk, _):
      off = (kk * NW + wid) * CHUNK
      pltpu.sync_copy(row_hbm.at[pl.ds(off, CHUNK)], idx_r)
      pltpu.sync_copy(col_hbm.at[pl.ds(off, CHUNK)], idx_c)
      pltpu.sync_copy(ew_hbm.at[pl.ds(off, CHUNK)], ew_v)
      pltpu.async_copy(xw_hbm.at[idx_r], rows_v, sem).wait()

      def norm_body(j, _):
        sl = pl.ds(j * L, L)
        dr = plsc.load_gather(dis_v, [idx_r[sl]])
        dc = plsc.load_gather(dis_v, [idx_c[sl]])
        norm_v[sl] = dr * ew_v[sl] * dc
        return ()

      lax.fori_loop(0, CHUNK // L, norm_body, ())

      def scale_body(g, _):
        nb = norm_v[pl.ds(g * L, L)]
        for rr in range(L):
          r = g * L + rr
          nv = jnp.full((L,), nb[rr], jnp.float32)
          for jj in range(D // L):
            sl = pl.ds(jj * L, L)
            rows_v[r, sl] = rows_v[r, sl] * nv
        return ()

      lax.fori_loop(0, CHUNK // L, scale_body, ())
      pltpu.sync_copy(rows_v, agg_acc.at[idx_c], add=True)
      return ()

    lax.fori_loop(0, n2, agg_body, ())
    plsc.subcore_barrier()

    sl = pl.ds(s * RPT, RPT)
    pltpu.sync_copy(agg_acc.at[sl], out_hbm.at[c, sl])

    @pl.when(s == NS - 1)
    def _():
      slr = pl.ds(NS * RPT, RPT_REM)
      pltpu.sync_copy(agg_acc.at[slr], out_hbm.at[c, slr])

  return k(xw, row, col, ew, zn, znd)


def kernel(x, edge_index, edge_weight, W, b, gamma, beta):
  row = edge_index[0].astype(jnp.int32)
  col = edge_index[1].astype(jnp.int32)
  ew = edge_weight.astype(jnp.float32)
  xw = _matmul_tc(x, W)
  zn = jnp.zeros((N,), jnp.float32)
  znd = jnp.zeros((N, D), jnp.float32)
  parts = _sc_aggregate(xw, row, col, ew, zn, znd)
  return _finish_tc(parts, x, b.reshape(1, D), gamma.reshape(1, D),
                    beta.reshape(1, D))
